# baseline (speedup 1.0000x reference)
import jax
import jax.numpy as jnp
from jax import lax
from jax.experimental import pallas as pl
from jax.experimental.pallas import tpu as pltpu
from jax.experimental.pallas import tpu_sc as plsc

DIM = 1024
ROWS = 16384
NC, NS, L = 2, 16, 16
NW = NC * NS
B_PER_W = ROWS // NW
CH = 8
N_CHUNKS = B_PER_W // CH
NSLOT = 6
PRE = 2
N_MAIN = (N_CHUNKS // NSLOT) * NSLOT


def _body(pos0_hbm, pos1_hbm, pe0_hbm, pe1_hbm, out_hbm,
          idx0_v, idx1_v, a_bufs, b_bufs, sg, so):
    wid = lax.axis_index("s") * NC + lax.axis_index("c")
    base = wid * B_PER_W
    pltpu.sync_copy(pos0_hbm.at[pl.ds(base, B_PER_W)], idx0_v)
    pltpu.sync_copy(pos1_hbm.at[pl.ds(base, B_PER_W)], idx1_v)

    def start_gather(c, slot):
        off = pl.multiple_of(c * CH, CH)
        pltpu.make_async_copy(
            pe0_hbm.at[idx0_v.at[pl.ds(off, CH)]], a_bufs[slot], sg[slot]).start()
        pltpu.make_async_copy(
            pe1_hbm.at[idx1_v.at[pl.ds(off, CH)]], b_bufs[slot], sg[slot]).start()

    def wait_gather(slot):
        pltpu.make_async_copy(
            pe0_hbm.at[idx0_v.at[pl.ds(0, CH)]], a_bufs[slot], sg[slot]).wait()
        pltpu.make_async_copy(
            pe1_hbm.at[idx1_v.at[pl.ds(0, CH)]], b_bufs[slot], sg[slot]).wait()

    def start_scatter(c, slot):
        off = pl.multiple_of(base + c * CH, CH)
        pltpu.make_async_copy(
            a_bufs[slot], out_hbm.at[pl.ds(off, CH)], so[slot]).start()

    def wait_scatter(slot):
        pltpu.make_async_copy(
            a_bufs[slot], out_hbm.at[pl.ds(0, CH)], so[slot]).wait()

    def add_chunk(slot):
        a, b = a_bufs[slot], b_bufs[slot]

        def add_row(r, _):
            for j in range(DIM // L):
                sl = pl.ds(j * L, L)
                plsc.addupdate(a.at[r, sl], b[r, sl])
            return 0
        lax.fori_loop(0, CH, add_row, 0, unroll=False)

    def step(c, slot):
        @pl.when(c >= NSLOT - PRE)
        def _():
            wait_scatter((slot + PRE) % NSLOT)

        @pl.when(c + PRE < N_CHUNKS)
        def _():
            start_gather(c + PRE, (slot + PRE) % NSLOT)

        wait_gather(slot)
        add_chunk(slot)
        start_scatter(c, slot)

    for p in range(PRE):
        start_gather(p, p)

    def group(g, _):
        for t in range(NSLOT):
            step(g * NSLOT + t, t)
        return 0

    lax.fori_loop(0, N_MAIN // NSLOT, group, 0, unroll=False)

    for c in range(N_MAIN, N_CHUNKS):
        step(jnp.int32(c), c % NSLOT)

    for c in range(N_CHUNKS - (NSLOT - PRE), N_CHUNKS):
        wait_scatter(c % NSLOT)


@jax.jit
def _run(pos0f, pos1f, pe0, pe1):
    mesh = plsc.VectorSubcoreMesh(core_axis_name="c", subcore_axis_name="s")

    def body(pos0r, pos1r, pe0r, pe1r, outr, idx0_v, idx1_v, *rest):
        a_bufs = rest[0:NSLOT]
        b_bufs = rest[NSLOT:2 * NSLOT]
        sg = rest[2 * NSLOT:3 * NSLOT]
        so = rest[3 * NSLOT:4 * NSLOT]
        _body(pos0r, pos1r, pe0r, pe1r, outr, idx0_v, idx1_v,
              a_bufs, b_bufs, sg, so)

    f = pl.kernel(
        body,
        out_type=jax.ShapeDtypeStruct((ROWS, DIM), jnp.float32),
        mesh=mesh,
        scratch_types=(
            [pltpu.VMEM((B_PER_W,), jnp.int32)] * 2
            + [pltpu.VMEM((CH, DIM), jnp.float32)] * (2 * NSLOT)
            + [pltpu.SemaphoreType.DMA] * (2 * NSLOT)
        ),
    )
    return f(pos0f, pos1f, pe0, pe1)


def kernel(pos0, pos1, pe0, pe1):
    batch, seq = pos0.shape
    pos0f = pos0.reshape(-1).astype(jnp.int32)
    pos1f = pos1.reshape(-1).astype(jnp.int32)
    out = _run(pos0f, pos1f, pe0, pe1)
    return out.reshape(batch, seq, DIM)


# speedup vs baseline: 1.0441x; 1.0441x over previous
import jax
import jax.numpy as jnp
from jax import lax
from jax.experimental import pallas as pl
from jax.experimental.pallas import tpu as pltpu
from jax.experimental.pallas import tpu_sc as plsc

DIM = 1024
ROWS = 16384
NC, NS, L = 2, 16, 16
NW = NC * NS
B_PER_W = ROWS // NW
CH = 8
N_CHUNKS = B_PER_W // CH
NSLOT = 4
PRE = 2
N_MAIN = (N_CHUNKS // NSLOT) * NSLOT


def _body(pos0_hbm, pos1_hbm, pe0_hbm, pe1_hbm, out_hbm,
          idx0_v, idx1_v, a_bufs, b_bufs, sg, so):
    wid = lax.axis_index("s") * NC + lax.axis_index("c")
    base = wid * B_PER_W
    pltpu.sync_copy(pos0_hbm.at[pl.ds(base, B_PER_W)], idx0_v)
    pltpu.sync_copy(pos1_hbm.at[pl.ds(base, B_PER_W)], idx1_v)

    def start_gather(c, slot):
        off = pl.multiple_of(c * CH, CH)
        pltpu.make_async_copy(
            pe0_hbm.at[idx0_v.at[pl.ds(off, CH)]], a_bufs[slot], sg[slot]).start()
        pltpu.make_async_copy(
            pe1_hbm.at[idx1_v.at[pl.ds(off, CH)]], b_bufs[slot], sg[slot]).start()

    def wait_gather(slot):
        pltpu.make_async_copy(
            pe0_hbm.at[idx0_v.at[pl.ds(0, CH)]], a_bufs[slot], sg[slot]).wait()
        pltpu.make_async_copy(
            pe1_hbm.at[idx1_v.at[pl.ds(0, CH)]], b_bufs[slot], sg[slot]).wait()

    def start_scatter(c, slot):
        off = pl.multiple_of(base + c * CH, CH)
        pltpu.make_async_copy(
            a_bufs[slot], out_hbm.at[pl.ds(off, CH)], so[slot]).start()

    def wait_scatter(slot):
        pltpu.make_async_copy(
            a_bufs[slot], out_hbm.at[pl.ds(0, CH)], so[slot]).wait()

    def add_chunk(slot):
        a, b = a_bufs[slot], b_bufs[slot]

        @plsc.parallel_loop(0, CH)
        def add_row(r):
            for j in range(DIM // L):
                sl = pl.ds(j * L, L)
                plsc.addupdate(a.at[r, sl], b[r, sl])

    def step(c, slot):
        @pl.when(c >= NSLOT - PRE)
        def _():
            wait_scatter((slot + PRE) % NSLOT)

        @pl.when(c + PRE < N_CHUNKS)
        def _():
            start_gather(c + PRE, (slot + PRE) % NSLOT)

        wait_gather(slot)
        add_chunk(slot)
        start_scatter(c, slot)

    for p in range(PRE):
        start_gather(p, p)

    def group(g, _):
        for t in range(NSLOT):
            step(g * NSLOT + t, t)
        return 0

    lax.fori_loop(0, N_MAIN // NSLOT, group, 0, unroll=False)

    for c in range(N_MAIN, N_CHUNKS):
        step(jnp.int32(c), c % NSLOT)

    for c in range(N_CHUNKS - (NSLOT - PRE), N_CHUNKS):
        wait_scatter(c % NSLOT)


@jax.jit
def _run(pos0f, pos1f, pe0, pe1):
    mesh = plsc.VectorSubcoreMesh(core_axis_name="c", subcore_axis_name="s")

    def body(pos0r, pos1r, pe0r, pe1r, outr, idx0_v, idx1_v, *rest):
        a_bufs = rest[0:NSLOT]
        b_bufs = rest[NSLOT:2 * NSLOT]
        sg = rest[2 * NSLOT:3 * NSLOT]
        so = rest[3 * NSLOT:4 * NSLOT]
        _body(pos0r, pos1r, pe0r, pe1r, outr, idx0_v, idx1_v,
              a_bufs, b_bufs, sg, so)

    f = pl.kernel(
        body,
        out_type=jax.ShapeDtypeStruct((ROWS, DIM), jnp.float32),
        mesh=mesh,
        scratch_types=(
            [pltpu.VMEM((B_PER_W,), jnp.int32)] * 2
            + [pltpu.VMEM((CH, DIM), jnp.float32)] * (2 * NSLOT)
            + [pltpu.SemaphoreType.DMA] * (2 * NSLOT)
        ),
    )
    return f(pos0f, pos1f, pe0, pe1)


def kernel(pos0, pos1, pe0, pe1):
    batch, seq = pos0.shape
    pos0f = pos0.reshape(-1).astype(jnp.int32)
    pos1f = pos1.reshape(-1).astype(jnp.int32)
    out = _run(pos0f, pos1f, pe0, pe1)
    return out.reshape(batch, seq, DIM)
